# baseline (speedup 1.0000x reference)
import functools

import jax
import jax.numpy as jnp
from jax import lax
from jax.experimental import pallas as pl
from jax.experimental.pallas import tpu as pltpu
from jax.experimental.pallas import tpu_sc as plsc

N = 10000
E = 160000
DF = 128
RPAD = 10112
NPAD = 10016
NROWS = N + 1
TROW = RPAD // 128
SIZE = RPAD * NROWS

NW = 32
CHUNK = 128
E_PAD = 163840
NCHUNK = E_PAD // (NW * CHUNK)
NFIRE = NCHUNK + 1
CAP = NFIRE * CHUNK
NBANDS = 5
HROWS = N // NBANDS
HSIZE = RPAD * (HROWS + 1)
HPAD_FIDX = HROWS * RPAD
HBLK = 40
HNBLK = HROWS // HBLK

BLK = 80
NBLK = N // BLK


def _st_body(x_ref, w_ref, at_ref, s_ref, t_ref):
    wx = lax.dot_general(x_ref[...], w_ref[...], (((1,), (1,)), ((), ())),
                         preferred_element_type=jnp.float32)
    a1 = at_ref[0, pl.ds(0, DF)]
    a2 = at_ref[0, pl.ds(DF, DF)]
    s = jnp.sum(wx * a1[None, :], axis=1)
    t = jnp.sum(wx * a2[None, :], axis=1)
    s_ref[0, pl.ds(0, N)] = s
    t_ref[0, pl.ds(0, N)] = t
    s_ref[0, pl.ds(N, NPAD - N)] = jnp.zeros((NPAD - N,), jnp.float32)
    t_ref[0, pl.ds(N, NPAD - N)] = jnp.zeros((NPAD - N,), jnp.float32)


_sc_mesh = plsc.VectorSubcoreMesh(core_axis_name="c", subcore_axis_name="s")


def _make_sc_scatter(b0):
    @functools.partial(
        pl.kernel,
        out_type=(),
        mesh=_sc_mesh,
        compiler_params=pltpu.CompilerParams(needs_layout_passes=False),
        scratch_types=[
            pltpu.VMEM_SHARED((NPAD,), jnp.float32),
            pltpu.VMEM_SHARED((NPAD,), jnp.float32),
            pltpu.VMEM((NPAD,), jnp.float32),
            pltpu.VMEM((NPAD,), jnp.float32),
            pltpu.VMEM((NCHUNK, CHUNK), jnp.int32),
            pltpu.VMEM((CAP,), jnp.int32),
            pltpu.VMEM((NFIRE, CHUNK), jnp.float32),
            pltpu.VMEM((NFIRE, CHUNK), jnp.int32),
            pltpu.SemaphoreType.DMA,
        ],
    )
    def sc_scatter(code_hbm, s_hbm, t_hbm, buf_ref,
                   sh_s, sh_t, sl_v, tl_v, codev, flt, cv, fv, sem):
        cid = lax.axis_index("c")
        sid = lax.axis_index("s")
        wid = sid * 2 + cid

        @pl.when(sid == 0)
        def _():
            pltpu.sync_copy(s_hbm, sh_s)
            pltpu.sync_copy(t_hbm, sh_t)

        plsc.subcore_barrier()
        pltpu.sync_copy(sh_s, sl_v)
        pltpu.sync_copy(sh_t, tl_v)
        pltpu.sync_copy(code_hbm.at[pl.ds(wid * NCHUNK, NCHUNK)], codev)

        lanes = lax.iota(jnp.int32, 16)

        def filt(c, p):
            for i in range(CHUNK // 16):
                v = codev[c, pl.ds(i * 16, 16)]
                sidx = v // N
                inb = (sidx >= b0) & (sidx < b0 + HROWS)
                plsc.store_compressed(flt.at[pl.ds(p, 16)], v, mask=inb)
                p = p + jnp.max(plsc.all_reduce_population_count(inb))
            return p

        n = lax.fori_loop(0, NCHUNK, filt, jnp.int32(0))

        def prefill(k, carry):
            fv[k >> 3, pl.ds((k & 7) * 16, 16)] = jnp.full(
                (16,), HPAD_FIDX, jnp.int32)
            return carry

        lax.fori_loop(0, NFIRE * 8, prefill, 0)

        def emit(j, carry):
            base = j * 16
            v = flt[pl.ds(base, 16)]
            tail = lanes < (n - base)
            vv = jnp.where(tail, v, 0)
            sidx = vv // N
            didx = vv - sidx * N
            z = plsc.load_gather(sl_v, [sidx]) + plsc.load_gather(tl_v, [didx])
            zlr = jnp.where(z >= 0.0, z, 0.1 * z)
            coef = jnp.exp(zlr)
            fidx = jnp.where(tail, vv + sidx * (RPAD - N) - b0 * RPAD,
                             HPAD_FIDX)
            row = j >> 3
            col = (j & 7) * 16
            cv[row, pl.ds(col, 16)] = coef
            fv[row, pl.ds(col, 16)] = fidx
            return carry

        lax.fori_loop(0, (n + 15) // 16, emit, 0)

        nf = (n + CHUNK - 1) // CHUNK

        def fire(c, carry):
            pltpu.async_copy(cv.at[c], buf_ref.at[fv.at[c]], sem)
            return carry

        lax.fori_loop(0, nf, fire, 0)

        def drain(c, carry):
            pltpu.make_async_copy(cv.at[c], buf_ref.at[fv.at[c]], sem).wait()
            return carry

        lax.fori_loop(0, nf, drain, 0)

    return sc_scatter


_sc_scatters = [_make_sc_scatter(k * HROWS) for k in range(NBANDS)]


def _norm_block(blk, g, row0):
    rs = jnp.sum(blk, axis=1)
    fix = (rs == 0.0).astype(jnp.float32)
    inv = 1.0 / (rs + fix)
    row_ids = lax.broadcasted_iota(jnp.int32, (HBLK, N), 0) + g * HBLK + row0
    col_ids = lax.broadcasted_iota(jnp.int32, (HBLK, N), 1)
    dmask = (col_ids == row_ids).astype(jnp.float32)
    core = lax.slice(blk, (0, 0), (HBLK, N))
    return (core + dmask * fix[:, None]) * inv[:, None]


def _make_norm_first(row0):
    def body(buf_ref, out_ref):
        g = pl.program_id(0)
        out_ref[...] = _norm_block(buf_ref[...], g, row0)

    return body


def _make_norm_next(row0):
    def body(buf_ref, prev_ref, out_ref):
        del prev_ref
        g = pl.program_id(0)
        out_ref[...] = _norm_block(buf_ref[...], g, row0)

    return body


def kernel(x, edge_index, W, a):
    at2d = a.reshape(1, 2 * DF)
    s2d, t2d = pl.pallas_call(
        _st_body,
        out_shape=[jax.ShapeDtypeStruct((1, NPAD), jnp.float32),
                   jax.ShapeDtypeStruct((1, NPAD), jnp.float32)],
    )(x, W, at2d)
    s1d = s2d.reshape(NPAD)
    t1d = t2d.reshape(NPAD)

    src = edge_index[0].astype(jnp.int32)
    dst = edge_index[1].astype(jnp.int32)
    npad = E_PAD - E
    code = src * N + dst
    code_p = jnp.concatenate([code, jnp.full((npad,), N * N, jnp.int32)])
    code_p = code_p.reshape(E_PAD // CHUNK, CHUNK)

    denses = []
    for k in range(NBANDS):
        buf = jax.new_ref(jnp.zeros((HSIZE,), jnp.float32))
        _sc_scatters[k](code_p, s1d, t1d, buf)
        denses.append(buf[...].reshape(HROWS + 1, RPAD))

    out = pl.pallas_call(
        _make_norm_first(0),
        grid=(HNBLK,),
        in_specs=[pl.BlockSpec((HBLK, RPAD), lambda g: (g, 0))],
        out_specs=pl.BlockSpec((HBLK, N), lambda g: (g, 0)),
        out_shape=jax.ShapeDtypeStruct((N, N), jnp.float32),
    )(denses[0])
    for k in range(1, NBANDS):
        out = pl.pallas_call(
            _make_norm_next(k * HROWS),
            grid=(HNBLK,),
            in_specs=[pl.BlockSpec((HBLK, RPAD), lambda g: (g, 0)),
                      pl.BlockSpec(memory_space=pl.ANY)],
            out_specs=pl.BlockSpec((HBLK, N),
                                   lambda g, k=k: (g + k * HNBLK, 0)),
            out_shape=jax.ShapeDtypeStruct((N, N), jnp.float32),
            input_output_aliases={1: 0},
        )(denses[k], out)
    return out


# speedup vs baseline: 1.6037x; 1.6037x over previous
import functools

import jax
import jax.numpy as jnp
from jax import lax
from jax.experimental import pallas as pl
from jax.experimental.pallas import tpu as pltpu
from jax.experimental.pallas import tpu_sc as plsc

N = 10000
E = 160000
DF = 128
RPAD = 10112
NPAD = 10016
NROWS = N + 1
TROW = RPAD // 128
SIZE = RPAD * NROWS

NW = 32
CHUNK = 128
E_PAD = 163840
NCHUNK = E_PAD // (NW * CHUNK)
NFIRE = NCHUNK + 1
CAP = NFIRE * CHUNK
NBANDS = 2
HROWS = N // NBANDS
HSIZE = RPAD * (HROWS + 1)
HPAD_FIDX = HROWS * RPAD
HBLK = 40
HNBLK = HROWS // HBLK

BLK = 80
NBLK = N // BLK


def _st_body(x_ref, w_ref, at_ref, s_ref, t_ref):
    wx = lax.dot_general(x_ref[...], w_ref[...], (((1,), (1,)), ((), ())),
                         preferred_element_type=jnp.float32)
    a1 = at_ref[0, pl.ds(0, DF)]
    a2 = at_ref[0, pl.ds(DF, DF)]
    s = jnp.sum(wx * a1[None, :], axis=1)
    t = jnp.sum(wx * a2[None, :], axis=1)
    s_ref[0, pl.ds(0, N)] = s
    t_ref[0, pl.ds(0, N)] = t
    s_ref[0, pl.ds(N, NPAD - N)] = jnp.zeros((NPAD - N,), jnp.float32)
    t_ref[0, pl.ds(N, NPAD - N)] = jnp.zeros((NPAD - N,), jnp.float32)


_sc_mesh = plsc.VectorSubcoreMesh(core_axis_name="c", subcore_axis_name="s")


def _make_sc_scatter(b0):
    @functools.partial(
        pl.kernel,
        out_type=(),
        mesh=_sc_mesh,
        compiler_params=pltpu.CompilerParams(needs_layout_passes=False),
        scratch_types=[
            pltpu.VMEM_SHARED((NPAD,), jnp.float32),
            pltpu.VMEM_SHARED((NPAD,), jnp.float32),
            pltpu.VMEM((NPAD,), jnp.float32),
            pltpu.VMEM((NPAD,), jnp.float32),
            pltpu.VMEM((NCHUNK, CHUNK), jnp.int32),
            pltpu.VMEM((CAP,), jnp.int32),
            pltpu.VMEM((NFIRE, CHUNK), jnp.float32),
            pltpu.VMEM((NFIRE, CHUNK), jnp.int32),
            pltpu.SemaphoreType.DMA,
        ],
    )
    def sc_scatter(code_hbm, s_hbm, t_hbm, buf_ref,
                   sh_s, sh_t, sl_v, tl_v, codev, flt, cv, fv, sem):
        cid = lax.axis_index("c")
        sid = lax.axis_index("s")
        wid = sid * 2 + cid

        @pl.when(sid == 0)
        def _():
            pltpu.sync_copy(s_hbm, sh_s)
            pltpu.sync_copy(t_hbm, sh_t)

        plsc.subcore_barrier()
        pltpu.sync_copy(sh_s, sl_v)
        pltpu.sync_copy(sh_t, tl_v)
        pltpu.sync_copy(code_hbm.at[pl.ds(wid * NCHUNK, NCHUNK)], codev)

        lanes = lax.iota(jnp.int32, 16)

        def filt(c, p):
            for i in range(CHUNK // 16):
                v = codev[c, pl.ds(i * 16, 16)]
                sidx = v // N
                inb = (sidx >= b0) & (sidx < b0 + HROWS)
                plsc.store_compressed(flt.at[pl.ds(p, 16)], v, mask=inb)
                p = p + jnp.max(plsc.all_reduce_population_count(inb))
            return p

        n = lax.fori_loop(0, NCHUNK, filt, jnp.int32(0))

        def prefill(k, carry):
            fv[k >> 3, pl.ds((k & 7) * 16, 16)] = jnp.full(
                (16,), HPAD_FIDX, jnp.int32)
            return carry

        lax.fori_loop(0, NFIRE * 8, prefill, 0)

        def emit(j, carry):
            base = j * 16
            v = flt[pl.ds(base, 16)]
            tail = lanes < (n - base)
            vv = jnp.where(tail, v, 0)
            sidx = vv // N
            didx = vv - sidx * N
            z = plsc.load_gather(sl_v, [sidx]) + plsc.load_gather(tl_v, [didx])
            zlr = jnp.where(z >= 0.0, z, 0.1 * z)
            coef = jnp.exp(zlr)
            fidx = jnp.where(tail, vv + sidx * (RPAD - N) - b0 * RPAD,
                             HPAD_FIDX)
            row = j >> 3
            col = (j & 7) * 16
            cv[row, pl.ds(col, 16)] = coef
            fv[row, pl.ds(col, 16)] = fidx
            return carry

        lax.fori_loop(0, (n + 15) // 16, emit, 0)

        nf = (n + CHUNK - 1) // CHUNK

        def fire(c, carry):
            pltpu.async_copy(cv.at[c], buf_ref.at[fv.at[c]], sem)
            return carry

        lax.fori_loop(0, nf, fire, 0)

        def drain(c, carry):
            pltpu.make_async_copy(cv.at[c], buf_ref.at[fv.at[c]], sem).wait()
            return carry

        lax.fori_loop(0, nf, drain, 0)

    return sc_scatter


_sc_scatters = [_make_sc_scatter(k * HROWS) for k in range(NBANDS)]


def _norm_block(blk, g, row0):
    rs = jnp.sum(blk, axis=1)
    fix = (rs == 0.0).astype(jnp.float32)
    inv = 1.0 / (rs + fix)
    row_ids = lax.broadcasted_iota(jnp.int32, (HBLK, N), 0) + g * HBLK + row0
    col_ids = lax.broadcasted_iota(jnp.int32, (HBLK, N), 1)
    dmask = (col_ids == row_ids).astype(jnp.float32)
    core = lax.slice(blk, (0, 0), (HBLK, N))
    return (core + dmask * fix[:, None]) * inv[:, None]


def _make_norm_first(row0):
    def body(buf_ref, out_ref):
        g = pl.program_id(0)
        out_ref[...] = _norm_block(buf_ref[...], g, row0)

    return body


def _make_norm_next(row0):
    def body(buf_ref, prev_ref, out_ref):
        del prev_ref
        g = pl.program_id(0)
        out_ref[...] = _norm_block(buf_ref[...], g, row0)

    return body


def kernel(x, edge_index, W, a):
    at2d = a.reshape(1, 2 * DF)
    s2d, t2d = pl.pallas_call(
        _st_body,
        out_shape=[jax.ShapeDtypeStruct((1, NPAD), jnp.float32),
                   jax.ShapeDtypeStruct((1, NPAD), jnp.float32)],
    )(x, W, at2d)
    s1d = s2d.reshape(NPAD)
    t1d = t2d.reshape(NPAD)

    src = edge_index[0].astype(jnp.int32)
    dst = edge_index[1].astype(jnp.int32)
    npad = E_PAD - E
    code = src * N + dst
    code_p = jnp.concatenate([code, jnp.full((npad,), N * N, jnp.int32)])
    code_p = code_p.reshape(E_PAD // CHUNK, CHUNK)

    denses = []
    for k in range(NBANDS):
        buf = jax.new_ref(jnp.zeros((HSIZE,), jnp.float32))
        _sc_scatters[k](code_p, s1d, t1d, buf)
        denses.append(buf[...].reshape(HROWS + 1, RPAD))

    out = pl.pallas_call(
        _make_norm_first(0),
        grid=(HNBLK,),
        in_specs=[pl.BlockSpec((HBLK, RPAD), lambda g: (g, 0))],
        out_specs=pl.BlockSpec((HBLK, N), lambda g: (g, 0)),
        out_shape=jax.ShapeDtypeStruct((N, N), jnp.float32),
    )(denses[0])
    for k in range(1, NBANDS):
        out = pl.pallas_call(
            _make_norm_next(k * HROWS),
            grid=(HNBLK,),
            in_specs=[pl.BlockSpec((HBLK, RPAD), lambda g: (g, 0)),
                      pl.BlockSpec(memory_space=pl.ANY)],
            out_specs=pl.BlockSpec((HBLK, N),
                                   lambda g, k=k: (g + k * HNBLK, 0)),
            out_shape=jax.ShapeDtypeStruct((N, N), jnp.float32),
            input_output_aliases={1: 0},
        )(denses[k], out)
    return out
